# baseline (speedup 1.0000x reference)
import functools

import jax
import jax.numpy as jnp
import numpy as np
from jax.experimental import pallas as pl
from jax.experimental.pallas import tpu as pltpu

_K = 64
_DIM = 128
_NITER = 25
_N = 4096

_INIT_ROWS = np.asarray(
    jax.random.permutation(jax.random.key(42), _N))[:_K]


def _bf16_split3(x):
    hi = x.astype(jnp.bfloat16)
    r1 = x - hi.astype(jnp.float32)
    mid = r1.astype(jnp.bfloat16)
    lo = (r1 - mid.astype(jnp.float32)).astype(jnp.bfloat16)
    return hi, mid, lo


def _body(emb_ref, out_ref, m2_ref):
    emb = emb_ref[...]
    cent0 = jnp.concatenate(
        [jax.lax.slice(emb, (int(r), 0), (int(r) + 1, _DIM))
         for r in _INIT_ROWS], axis=0)
    emb_t = jnp.swapaxes(emb, 0, 1)
    emb_t16 = emb_t.astype(jnp.bfloat16)
    e_hi, e_mid, e_lo = _bf16_split3(emb)
    e_hi_aug = jnp.concatenate(
        [e_hi, jnp.ones((_N, 1), jnp.bfloat16)], axis=1)
    enorm = jnp.swapaxes(
        jnp.sum(emb * emb, axis=1, keepdims=True), 0, 1)
    rows = jax.lax.broadcasted_iota(
        jnp.int32, (_K, _N), 0).astype(jnp.float32)

    def dists_t(cent):
        cnorm = jnp.sum(cent * cent, axis=1, keepdims=True)
        g = jax.lax.dot_general(
            cent.astype(jnp.bfloat16), emb_t16, (((1,), (0,)), ((), ())),
            preferred_element_type=jnp.float32)
        return enorm - 2.0 * g + cnorm

    def first_eq_idx(x, target):
        return jnp.min(jnp.where(x == target, rows, float(_K)),
                       axis=0, keepdims=True)

    def step(_, cent):
        sq = dists_t(cent)
        mn = jnp.min(sq, axis=0, keepdims=True)
        idx = first_eq_idx(sq, mn)
        oh16 = (idx == rows).astype(jnp.bfloat16)
        dims = (((1,), (0,)), ((), ()))
        t = jax.lax.dot_general(
            oh16, e_hi_aug, dims,
            preferred_element_type=jnp.float32)
        sums = (jax.lax.slice(t, (0, 0), (_K, _DIM))
                + jax.lax.dot_general(
                    oh16, e_mid, dims, preferred_element_type=jnp.float32)
                + jax.lax.dot_general(
                    oh16, e_lo, dims, preferred_element_type=jnp.float32))
        counts = jax.lax.slice(t, (0, _DIM), (_K, _DIM + 1))
        return jnp.where(counts > 0, sums / jnp.maximum(counts, 1.0), cent)

    cent = jax.lax.fori_loop(0, _NITER, step, cent0, unroll=5)

    sq = dists_t(cent)
    dist = jnp.sqrt(jnp.maximum(sq, 0.0))
    mx = jnp.max(dist, axis=0, keepdims=True)
    m1 = first_eq_idx(dist, mx)
    dist2 = jnp.where(rows == m1, -jnp.inf, dist)
    mx2 = jnp.max(dist2, axis=0, keepdims=True)
    m2 = first_eq_idx(dist2, mx2)
    out_ref[...] = cent
    m2_ref[...] = m2.astype(jnp.int32)


def _sc_gather(table, idx):
    from jax.experimental.pallas import tpu_sc as plsc

    info = plsc.get_sparse_core_info()
    nw = info.num_cores * info.num_subcores
    b_per_w = _N // nw
    mesh = plsc.VectorSubcoreMesh(core_axis_name="c", subcore_axis_name="s")

    @functools.partial(
        pl.kernel, mesh=mesh,
        out_type=jax.ShapeDtypeStruct((_N, _DIM), jnp.float32),
        scratch_types=[
            pltpu.VMEM((b_per_w,), jnp.int32),
            pltpu.VMEM((b_per_w, _DIM), jnp.float32),
            pltpu.SemaphoreType.DMA,
        ],
    )
    def k(table_hbm, idx_hbm, out_hbm, idx_v, rows_v, sem):
        wid = jax.lax.axis_index("s") * info.num_cores + jax.lax.axis_index("c")
        base = wid * b_per_w
        pltpu.sync_copy(idx_hbm.at[pl.ds(base, b_per_w)], idx_v)
        pltpu.async_copy(table_hbm.at[idx_v], rows_v, sem).wait()
        pltpu.sync_copy(rows_v, out_hbm.at[pl.ds(base, b_per_w)])

    return k(table, idx)


def kernel(embeddings, batch_id):
    del batch_id
    cent, m2 = pl.pallas_call(
        _body,
        out_shape=[jax.ShapeDtypeStruct((_K, _DIM), jnp.float32),
                   jax.ShapeDtypeStruct((1, _N), jnp.int32)],
    )(embeddings)
    return _sc_gather(cent, m2.reshape(_N))


# speedup vs baseline: 3.1798x; 3.1798x over previous
import jax
import jax.numpy as jnp
import numpy as np
from jax.experimental import pallas as pl
from jax.experimental.pallas import tpu as pltpu

_K = 64
_DIM = 128
_NITER = 25
_N = 4096

_INIT_ROWS = np.asarray(
    jax.random.permutation(jax.random.key(42), _N))[:_K]


def _bf16_split3(x):
    hi = x.astype(jnp.bfloat16)
    r1 = x - hi.astype(jnp.float32)
    mid = r1.astype(jnp.bfloat16)
    lo = (r1 - mid.astype(jnp.float32)).astype(jnp.bfloat16)
    return hi, mid, lo


def _body(emb_ref, out_ref):
    emb = emb_ref[...]
    cent0 = jnp.concatenate(
        [jax.lax.slice(emb, (int(r), 0), (int(r) + 1, _DIM))
         for r in _INIT_ROWS], axis=0)
    emb_t = jnp.swapaxes(emb, 0, 1)
    emb_t16 = emb_t.astype(jnp.bfloat16)
    e_hi, e_mid, e_lo = _bf16_split3(emb)
    e_hi_aug = jnp.concatenate(
        [e_hi, jnp.ones((_N, 1), jnp.bfloat16)], axis=1)
    enorm = jnp.swapaxes(
        jnp.sum(emb * emb, axis=1, keepdims=True), 0, 1)
    rows = jax.lax.broadcasted_iota(
        jnp.int32, (_K, _N), 0).astype(jnp.float32)

    def dists_t(cent):
        cnorm = jnp.sum(cent * cent, axis=1, keepdims=True)
        g = jax.lax.dot_general(
            cent.astype(jnp.bfloat16), emb_t16, (((1,), (0,)), ((), ())),
            preferred_element_type=jnp.float32)
        return enorm - 2.0 * g + cnorm

    def first_eq_idx(x, target):
        return jnp.min(jnp.where(x == target, rows, float(_K)),
                       axis=0, keepdims=True)

    def step(_, cent):
        sq = dists_t(cent)
        mn = jnp.min(sq, axis=0, keepdims=True)
        idx = first_eq_idx(sq, mn)
        oh16 = (idx == rows).astype(jnp.bfloat16)
        dims = (((1,), (0,)), ((), ()))
        t = jax.lax.dot_general(
            oh16, e_hi_aug, dims,
            preferred_element_type=jnp.float32)
        sums = (jax.lax.slice(t, (0, 0), (_K, _DIM))
                + jax.lax.dot_general(
                    oh16, e_mid, dims, preferred_element_type=jnp.float32)
                + jax.lax.dot_general(
                    oh16, e_lo, dims, preferred_element_type=jnp.float32))
        counts = jax.lax.slice(t, (0, _DIM), (_K, _DIM + 1))
        return jnp.where(counts > 0, sums / jnp.maximum(counts, 1.0), cent)

    cent = jax.lax.fori_loop(0, _NITER, step, cent0, unroll=5)

    sq = dists_t(cent)
    dist = jnp.sqrt(jnp.maximum(sq, 0.0))
    mx = jnp.max(dist, axis=0, keepdims=True)
    m1 = first_eq_idx(dist, mx)
    dist2 = jnp.where(rows == m1, -jnp.inf, dist)
    mx2 = jnp.max(dist2, axis=0, keepdims=True)
    m2 = first_eq_idx(dist2, mx2)
    oh2_t16 = (m2 == rows).astype(jnp.bfloat16)
    c_hi, c_mid, c_lo = _bf16_split3(jnp.swapaxes(cent, 0, 1))
    dims = (((1,), (0,)), ((), ()))
    out_t = (jax.lax.dot_general(
                 c_hi, oh2_t16, dims, preferred_element_type=jnp.float32)
             + jax.lax.dot_general(
                 c_mid, oh2_t16, dims, preferred_element_type=jnp.float32)
             + jax.lax.dot_general(
                 c_lo, oh2_t16, dims, preferred_element_type=jnp.float32))
    out_ref[...] = jnp.swapaxes(out_t, 0, 1)


def kernel(embeddings, batch_id):
    del batch_id
    return pl.pallas_call(
        _body,
        out_shape=jax.ShapeDtypeStruct((_N, _DIM), jnp.float32),
    )(embeddings)
